# Initial kernel scaffold; baseline (speedup 1.0000x reference)
import jax
import jax.numpy as jnp
from jax.experimental import pallas as pl


def kernel(x, edge_index, W1, att_src1, att_dst1, bias1, gamma, beta, W2, att_src2, att_dst2, bias2):
    raise NotImplementedError("write your pallas kernel here")


# speedup vs baseline: 40.7780x; 40.7780x over previous
import functools

import jax
import jax.numpy as jnp
from jax import lax
from jax.experimental import pallas as pl
from jax.experimental.pallas import tpu as pltpu
from jax.experimental.pallas import tpu_sc as plsc

N = 10000
E = 320000
IN_DIM = 128
HID = 128
LAT = 30
LATP = 32
W = 32
FP1 = HID // W

NC = 2
NS = 16
NW = NC * NS
EPT = E // NW
K = 80
NB = EPT // K
RING = 5
RPT = 624
ZR = 16


def _lrelu(v):
    return jnp.maximum(v, 0.2 * v)


def _make_edge_kernel(FP):
    mesh = plsc.VectorSubcoreMesh(
        core_axis_name="c", subcore_axis_name="s", num_cores=NC, num_subcores=NS
    )

    @functools.partial(
        pl.kernel,
        out_type=(
            jax.ShapeDtypeStruct((FP, NC, N, W), jnp.float32),
            jax.ShapeDtypeStruct((NW * N,), jnp.float32),
        ),
        mesh=mesh,
        scratch_types=[
            pltpu.VMEM((NB, K), jnp.int32),
            pltpu.VMEM((NB, K), jnp.int32),
            pltpu.VMEM((N,), jnp.float32),
            pltpu.VMEM((N,), jnp.float32),
            pltpu.VMEM((N,), jnp.float32),
            pltpu.VMEM((N,), jnp.float32),
            pltpu.VMEM((RING, K, W), jnp.float32),
            pltpu.VMEM((NB, K), jnp.float32),
            pltpu.VMEM((ZR, W), jnp.float32),
            pltpu.VMEM_SHARED((N, W), jnp.float32),
        ]
        + [pltpu.SemaphoreType.DMA] * RING
        + [pltpu.SemaphoreType.DMA] * RING,
        compiler_params=pltpu.CompilerParams(
            needs_layout_passes=False, use_tc_tiling_on_sc=False
        ),
    )
    def edge_kernel(edges_hbm, asrc_hbm, adst_hbm, hcat_hbm, out_hbm, den_hbm,
                    src_idx, dst_idx, asrc_v, adst_v, bsh_v, den_v,
                    rows, pstore, zbuf, acc, *sems):
        gsems = sems[:RING]
        ssems = sems[RING:]

        cid = lax.axis_index("c")
        sid = lax.axis_index("s")
        wid = cid * NS + sid

        pltpu.sync_copy(edges_hbm.at[0, wid], src_idx)
        pltpu.sync_copy(edges_hbm.at[1, wid], dst_idx)
        pltpu.sync_copy(asrc_hbm, asrc_v)
        pltpu.sync_copy(adst_hbm, adst_v)

        zero16 = jnp.zeros((16,), jnp.float32)

        def _amax(i, m):
            return jnp.maximum(m, asrc_v[pl.ds(i * 16, 16)])

        mx = jnp.max(lax.fori_loop(0, N // 16, _amax,
                                   jnp.full((16,), -3.4e38, jnp.float32)))

        def _bfill(i, _):
            bsh_v[pl.ds(i * 16, 16)] = _lrelu(mx + adst_v[pl.ds(i * 16, 16)])
            return 0

        lax.fori_loop(0, N // 16, _bfill, 0)

        def _zden(i, _):
            den_v[pl.ds(i * 16, 16)] = zero16
            return 0

        lax.fori_loop(0, N // 16, _zden, 0)

        def _zzb(r, _):
            for c0 in range(W // 16):
                zbuf[r, pl.ds(c0 * 16, 16)] = zero16
            return 0

        lax.fori_loop(0, ZR, _zzb, 0)

        row_base = sid * RPT
        nq = jnp.where(sid == NS - 1, (N - (NS - 1) * RPT) // ZR, RPT // ZR)

        for f in range(FP):
            h_hbm = hcat_hbm.at[f]

            for u in range(RING):
                pltpu.async_copy(h_hbm.at[src_idx.at[u]], rows.at[u], gsems[u])

            def _zacc(q, _):
                pltpu.sync_copy(zbuf, acc.at[pl.ds(row_base + q * ZR, ZR)])
                return 0

            lax.fori_loop(0, nq, _zacc, 0)
            plsc.subcore_barrier()

            def _outer(i, _):
                for u in range(RING):
                    b = i * RING + u
                    pltpu.make_async_copy(
                        h_hbm.at[src_idx.at[b]], rows.at[u], gsems[u]
                    ).wait()

                    if f == 0:
                        for g in range(K // 16):
                            s16 = src_idx[b, pl.ds(g * 16, 16)]
                            d16 = dst_idx[b, pl.ds(g * 16, 16)]
                            a_s = plsc.load_gather(asrc_v, [s16])
                            a_d = plsc.load_gather(adst_v, [d16])
                            bb = plsc.load_gather(bsh_v, [d16])
                            p = jnp.exp(_lrelu(a_s + a_d) - bb)
                            plsc.addupdate_scatter(den_v, [d16], p)
                            pstore[b, pl.ds(g * 16, 16)] = p

                    ru = rows.at[u]

                    def _scale(g2, _):
                        pg = pstore[b, pl.ds(g2 * 16, 16)]
                        base = g2 * 16
                        for lane in range(16):
                            pk = pg[lane]
                            for c0 in range(W // 16):
                                ru[base + lane, pl.ds(c0 * 16, 16)] = (
                                    ru[base + lane, pl.ds(c0 * 16, 16)] * pk
                                )
                        return 0

                    lax.fori_loop(0, K // 16, _scale, 0)

                    pltpu.async_copy(
                        rows.at[u], acc.at[dst_idx.at[b]], ssems[u], add=True
                    )

                    u3 = (u + 3) % RING
                    bn = b + 3

                    @pl.when(jnp.logical_and(bn >= RING, bn < NB))
                    def _():
                        pltpu.make_async_copy(
                            rows.at[u3], acc.at[dst_idx.at[bn - RING]],
                            ssems[u3]
                        ).wait()
                        pltpu.async_copy(
                            h_hbm.at[src_idx.at[bn]], rows.at[u3], gsems[u3]
                        )

                return 0

            lax.fori_loop(0, NB // RING, _outer, 0)

            for u in range(RING):
                b = NB - RING + u
                pltpu.make_async_copy(
                    rows.at[u], acc.at[dst_idx.at[b]], ssems[u]
                ).wait()
            plsc.subcore_barrier()

            def _wout(q, _):
                pltpu.sync_copy(
                    acc.at[pl.ds(row_base + q * ZR, ZR)],
                    out_hbm.at[f, cid, pl.ds(row_base + q * ZR, ZR)],
                )
                return 0

            lax.fori_loop(0, nq, _wout, 0)

        pltpu.sync_copy(den_v, den_hbm.at[pl.ds(wid * N, N)])

    return edge_kernel


_edge_l1 = _make_edge_kernel(FP1)
_edge_l2 = _make_edge_kernel(1)


def _tc1_body(x_ref, w_ref, asv_ref, adv_ref, hcat_ref, as_ref, ad_ref):
    x = x_ref[...]
    a_s = jnp.zeros((N,), jnp.float32)
    a_d = jnp.zeros((N,), jnp.float32)
    for cc in range(FP1):
        hc = jnp.dot(x, w_ref[:, cc * W:(cc + 1) * W],
                     preferred_element_type=jnp.float32)
        hcat_ref[cc] = hc
        a_s = a_s + jnp.sum(hc * asv_ref[:, cc * W:(cc + 1) * W], axis=1)
        a_d = a_d + jnp.sum(hc * adv_ref[:, cc * W:(cc + 1) * W], axis=1)
    as_ref[...] = a_s
    ad_ref[...] = a_d


def _tc2_body(o_ref, den_ref, hc_ref, as1_ref, ad1_ref,
              gamma_ref, beta_ref, bias1_ref, w2_ref, as2v_ref, ad2v_ref,
              h2_ref, as2_ref, ad2_ref):
    cc = pl.program_id(0)
    a_s1 = as1_ref[...]
    a_d1 = ad1_ref[...]
    b1 = _lrelu(jnp.max(a_s1) + a_d1)
    dt = jnp.sum(den_ref[...], axis=0)
    ps = jnp.exp(_lrelu(a_s1 + a_d1) - b1)
    dtot = (dt + ps + 1e-16)[:, None]
    o_cc = jnp.sum(o_ref[...], axis=(0, 1))
    hc = jnp.squeeze(hc_ref[...], axis=0)
    h_cc = (o_cc + ps[:, None] * hc) / dtot + bias1_ref[0]
    mu = jnp.mean(h_cc, axis=0)
    var = jnp.mean((h_cc - mu) ** 2, axis=0)
    hn = (h_cc - mu) / jnp.sqrt(var + 1e-5) * gamma_ref[0] + beta_ref[0]
    he = jnp.where(hn > 0, hn, jnp.exp(hn) - 1.0)
    h2p = jnp.dot(he, w2_ref[...], preferred_element_type=jnp.float32)

    @pl.when(cc == 0)
    def _():
        h2_ref[...] = h2p

    @pl.when(cc > 0)
    def _():
        h2_ref[...] = h2_ref[...] + h2p

    @pl.when(cc == FP1 - 1)
    def _():
        h2f = h2_ref[...]
        as2_ref[...] = jnp.sum(h2f * as2v_ref[...], axis=1)
        ad2_ref[...] = jnp.sum(h2f * ad2v_ref[...], axis=1)


def _tc3_body(out2p_ref, den2p_ref, h2_ref, as2_ref, ad2_ref,
              bias2_ref, z_ref):
    o = jnp.sum(out2p_ref[...], axis=(0, 1))
    a_s2 = as2_ref[...]
    a_d2 = ad2_ref[...]
    b2 = _lrelu(jnp.max(a_s2) + a_d2)
    dt = jnp.sum(den2p_ref[...], axis=0)
    ps = jnp.exp(_lrelu(a_s2 + a_d2) - b2)
    dtot = (dt + ps + 1e-16)[:, None]
    z_ref[...] = (o + ps[:, None] * h2_ref[...]) / dtot + bias2_ref[...]


_tc1 = pl.pallas_call(
    _tc1_body,
    out_shape=(
        jax.ShapeDtypeStruct((FP1, N, W), jnp.float32),
        jax.ShapeDtypeStruct((N,), jnp.float32),
        jax.ShapeDtypeStruct((N,), jnp.float32),
    ),
)

_tc2 = pl.pallas_call(
    _tc2_body,
    grid=(FP1,),
    in_specs=[
        pl.BlockSpec((1, NC, N, W), lambda cc: (cc, 0, 0, 0)),
        pl.BlockSpec((NW, N), lambda cc: (0, 0)),
        pl.BlockSpec((1, N, W), lambda cc: (cc, 0, 0)),
        pl.BlockSpec((N,), lambda cc: (0,)),
        pl.BlockSpec((N,), lambda cc: (0,)),
        pl.BlockSpec((1, 1, W), lambda cc: (cc, 0, 0)),
        pl.BlockSpec((1, 1, W), lambda cc: (cc, 0, 0)),
        pl.BlockSpec((1, 1, W), lambda cc: (cc, 0, 0)),
        pl.BlockSpec((W, LATP), lambda cc: (cc, 0)),
        pl.BlockSpec((1, LATP), lambda cc: (0, 0)),
        pl.BlockSpec((1, LATP), lambda cc: (0, 0)),
    ],
    out_specs=[
        pl.BlockSpec((N, LATP), lambda cc: (0, 0)),
        pl.BlockSpec((N,), lambda cc: (0,)),
        pl.BlockSpec((N,), lambda cc: (0,)),
    ],
    out_shape=(
        jax.ShapeDtypeStruct((N, LATP), jnp.float32),
        jax.ShapeDtypeStruct((N,), jnp.float32),
        jax.ShapeDtypeStruct((N,), jnp.float32),
    ),
)

_tc3 = pl.pallas_call(
    _tc3_body,
    out_shape=jax.ShapeDtypeStruct((N, LATP), jnp.float32),
)


def kernel(x, edge_index, W1, att_src1, att_dst1, bias1, gamma, beta,
           W2, att_src2, att_dst2, bias2):
    edges = edge_index.reshape(2, NW, NB, K)

    hcat, as1, ad1 = _tc1(x, W1, att_src1, att_dst1)
    out1, den1p = _edge_l1(edges, as1, ad1, hcat)

    w2p = jnp.pad(W2, ((0, 0), (0, LATP - LAT)))
    as2v = jnp.pad(att_src2, ((0, 0), (0, LATP - LAT)))
    ad2v = jnp.pad(att_dst2, ((0, 0), (0, LATP - LAT)))
    bias2p = jnp.pad(bias2, (0, LATP - LAT))

    h2, as2, ad2 = _tc2(out1, den1p.reshape(NW, N), hcat, as1, ad1,
                        gamma.reshape(FP1, 1, W), beta.reshape(FP1, 1, W),
                        bias1.reshape(FP1, 1, W), w2p, as2v, ad2v)
    out2, den2p = _edge_l2(edges, as2, ad2, h2.reshape(1, N, LATP))
    z = _tc3(out2, den2p.reshape(NW, N), h2, as2, ad2, bias2p)
    return z[:, :LAT]


# speedup vs baseline: 47.3525x; 1.1612x over previous
import functools

import jax
import jax.numpy as jnp
from jax import lax
from jax.experimental import pallas as pl
from jax.experimental.pallas import tpu as pltpu
from jax.experimental.pallas import tpu_sc as plsc

N = 10000
E = 320000
IN_DIM = 128
HID = 128
LAT = 30
LATP = 32
W = 32
FP1 = HID // W

NC = 2
NS = 16
NW = NC * NS
EPT = E // NW
K = 80
NB = EPT // K
RING = 5
RPT = 624
ZR = 16


def _lrelu(v):
    return jnp.maximum(v, 0.2 * v)


def _make_edge_kernel(FP):
    mesh = plsc.VectorSubcoreMesh(
        core_axis_name="c", subcore_axis_name="s", num_cores=NC, num_subcores=NS
    )

    @functools.partial(
        pl.kernel,
        out_type=(
            jax.ShapeDtypeStruct((FP, NC, N, W), jnp.float32),
            jax.ShapeDtypeStruct((NW * N,), jnp.float32),
        ),
        mesh=mesh,
        scratch_types=[
            pltpu.VMEM((NB, K), jnp.int32),
            pltpu.VMEM((NB, K), jnp.int32),
            pltpu.VMEM((N,), jnp.float32),
            pltpu.VMEM((N,), jnp.float32),
            pltpu.VMEM((N,), jnp.float32),
            pltpu.VMEM((RING, K, W), jnp.float32),
            pltpu.VMEM((RING, K, W), jnp.float32),
            pltpu.VMEM((NB, K), jnp.float32),
            pltpu.VMEM((ZR, W), jnp.float32),
            pltpu.VMEM_SHARED((N, W), jnp.float32),
        ]
        + [pltpu.SemaphoreType.DMA] * RING
        + [pltpu.SemaphoreType.DMA] * RING,
        compiler_params=pltpu.CompilerParams(
            needs_layout_passes=False, use_tc_tiling_on_sc=False
        ),
    )
    def edge_kernel(edges_hbm, asrc_hbm, adst_hbm, hcat_hbm, out_hbm, den_hbm,
                    src_idx, dst_idx, asrc_v, adst_v, den_v,
                    rows_g, rows_s, pstore, zbuf, acc, *sems):
        gsems = sems[:RING]
        ssems = sems[RING:]

        cid = lax.axis_index("c")
        sid = lax.axis_index("s")
        wid = cid * NS + sid

        pltpu.sync_copy(edges_hbm.at[0, wid], src_idx)
        pltpu.sync_copy(edges_hbm.at[1, wid], dst_idx)
        pltpu.sync_copy(asrc_hbm, asrc_v)
        pltpu.sync_copy(adst_hbm, adst_v)

        zero16 = jnp.zeros((16,), jnp.float32)

        def _amax(i, m):
            return jnp.maximum(m, asrc_v[pl.ds(i * 16, 16)])

        mx = jnp.max(lax.fori_loop(0, N // 16, _amax,
                                   jnp.full((16,), -3.4e38, jnp.float32)))

        def _zden(i, _):
            den_v[pl.ds(i * 16, 16)] = zero16
            return 0

        lax.fori_loop(0, N // 16, _zden, 0)

        def _zzb(r, _):
            for c0 in range(W // 16):
                zbuf[r, pl.ds(c0 * 16, 16)] = zero16
            return 0

        lax.fori_loop(0, ZR, _zzb, 0)

        row_base = sid * RPT
        nq = jnp.where(sid == NS - 1, (N - (NS - 1) * RPT) // ZR, RPT // ZR)

        for f in range(FP):
            h_hbm = hcat_hbm.at[f]

            for u in range(RING):
                pltpu.async_copy(h_hbm.at[src_idx.at[u]], rows_g.at[u],
                                 gsems[u])

            def _zacc(q, _):
                pltpu.sync_copy(zbuf, acc.at[pl.ds(row_base + q * ZR, ZR)])
                return 0

            lax.fori_loop(0, nq, _zacc, 0)
            plsc.subcore_barrier()

            def _outer(i, _):
                for u in range(RING):
                    b = i * RING + u
                    pltpu.make_async_copy(
                        h_hbm.at[src_idx.at[b]], rows_g.at[u], gsems[u]
                    ).wait()

                    if f == 0:
                        for g in range(K // 16):
                            s16 = src_idx[b, pl.ds(g * 16, 16)]
                            d16 = dst_idx[b, pl.ds(g * 16, 16)]
                            a_s = plsc.load_gather(asrc_v, [s16])
                            a_d = plsc.load_gather(adst_v, [d16])
                            p = jnp.exp(_lrelu(a_s + a_d) - _lrelu(mx + a_d))
                            plsc.addupdate_scatter(den_v, [d16], p)
                            pstore[b, pl.ds(g * 16, 16)] = p

                    @pl.when(b >= RING)
                    def _():
                        pltpu.make_async_copy(
                            rows_s.at[u], acc.at[dst_idx.at[b - RING]],
                            ssems[u]
                        ).wait()

                    rg = rows_g.at[u]
                    rs = rows_s.at[u]

                    def _scale(g2, _):
                        pg = pstore[b, pl.ds(g2 * 16, 16)]
                        base = g2 * 16
                        for lane in range(16):
                            pk = pg[lane]
                            for c0 in range(W // 16):
                                rs[base + lane, pl.ds(c0 * 16, 16)] = (
                                    rg[base + lane, pl.ds(c0 * 16, 16)] * pk
                                )
                        return 0

                    lax.fori_loop(0, K // 16, _scale, 0)

                    pltpu.async_copy(
                        rows_s.at[u], acc.at[dst_idx.at[b]], ssems[u], add=True
                    )

                    bn = b + RING

                    @pl.when(bn < NB)
                    def _():
                        pltpu.async_copy(
                            h_hbm.at[src_idx.at[bn]], rows_g.at[u], gsems[u]
                        )

                return 0

            lax.fori_loop(0, NB // RING, _outer, 0)

            for u in range(RING):
                b = NB - RING + u
                pltpu.make_async_copy(
                    rows_s.at[u], acc.at[dst_idx.at[b]], ssems[u]
                ).wait()
            plsc.subcore_barrier()

            def _wout(q, _):
                pltpu.sync_copy(
                    acc.at[pl.ds(row_base + q * ZR, ZR)],
                    out_hbm.at[f, cid, pl.ds(row_base + q * ZR, ZR)],
                )
                return 0

            lax.fori_loop(0, nq, _wout, 0)

        pltpu.sync_copy(den_v, den_hbm.at[pl.ds(wid * N, N)])

    return edge_kernel


_edge_l1 = _make_edge_kernel(FP1)
_edge_l2 = _make_edge_kernel(1)


def _tc1_body(x_ref, w_ref, asv_ref, adv_ref, hcat_ref, as_ref, ad_ref):
    h = jnp.dot(x_ref[...], w_ref[...], preferred_element_type=jnp.float32)
    for cc in range(FP1):
        hcat_ref[cc] = h[:, cc * W:(cc + 1) * W]
    as_ref[...] = jnp.sum(h * asv_ref[...], axis=1)
    ad_ref[...] = jnp.sum(h * adv_ref[...], axis=1)


def _tc2_body(o_ref, den_ref, hc_ref, as1_ref, ad1_ref,
              gamma_ref, beta_ref, bias1_ref, w2_ref, as2v_ref, ad2v_ref,
              h2_ref, as2_ref, ad2_ref):
    cc = pl.program_id(0)
    a_s1 = as1_ref[...]
    a_d1 = ad1_ref[...]
    b1 = _lrelu(jnp.max(a_s1) + a_d1)
    dt = jnp.sum(den_ref[...], axis=0)
    ps = jnp.exp(_lrelu(a_s1 + a_d1) - b1)
    dtot = (dt + ps + 1e-16)[:, None]
    o_cc = jnp.sum(o_ref[...], axis=(0, 1))
    hc = jnp.squeeze(hc_ref[...], axis=0)
    h_cc = (o_cc + ps[:, None] * hc) / dtot + bias1_ref[0]
    mu = jnp.mean(h_cc, axis=0)
    var = jnp.mean((h_cc - mu) ** 2, axis=0)
    hn = (h_cc - mu) / jnp.sqrt(var + 1e-5) * gamma_ref[0] + beta_ref[0]
    he = jnp.where(hn > 0, hn, jnp.exp(hn) - 1.0)
    h2p = jnp.dot(he, w2_ref[...], preferred_element_type=jnp.float32)

    @pl.when(cc == 0)
    def _():
        h2_ref[...] = h2p

    @pl.when(cc > 0)
    def _():
        h2_ref[...] = h2_ref[...] + h2p

    @pl.when(cc == FP1 - 1)
    def _():
        h2f = h2_ref[...]
        as2_ref[...] = jnp.sum(h2f * as2v_ref[...], axis=1)
        ad2_ref[...] = jnp.sum(h2f * ad2v_ref[...], axis=1)


def _tc3_body(out2p_ref, den2p_ref, h2_ref, as2_ref, ad2_ref,
              bias2_ref, z_ref):
    o = jnp.sum(out2p_ref[...], axis=(0, 1))
    a_s2 = as2_ref[...]
    a_d2 = ad2_ref[...]
    b2 = _lrelu(jnp.max(a_s2) + a_d2)
    dt = jnp.sum(den2p_ref[...], axis=0)
    ps = jnp.exp(_lrelu(a_s2 + a_d2) - b2)
    dtot = (dt + ps + 1e-16)[:, None]
    z_ref[...] = (o + ps[:, None] * h2_ref[...]) / dtot + bias2_ref[...]


_tc1 = pl.pallas_call(
    _tc1_body,
    out_shape=(
        jax.ShapeDtypeStruct((FP1, N, W), jnp.float32),
        jax.ShapeDtypeStruct((N,), jnp.float32),
        jax.ShapeDtypeStruct((N,), jnp.float32),
    ),
)

_tc2 = pl.pallas_call(
    _tc2_body,
    grid=(FP1,),
    in_specs=[
        pl.BlockSpec((1, NC, N, W), lambda cc: (cc, 0, 0, 0)),
        pl.BlockSpec((NW, N), lambda cc: (0, 0)),
        pl.BlockSpec((1, N, W), lambda cc: (cc, 0, 0)),
        pl.BlockSpec((N,), lambda cc: (0,)),
        pl.BlockSpec((N,), lambda cc: (0,)),
        pl.BlockSpec((1, 1, W), lambda cc: (cc, 0, 0)),
        pl.BlockSpec((1, 1, W), lambda cc: (cc, 0, 0)),
        pl.BlockSpec((1, 1, W), lambda cc: (cc, 0, 0)),
        pl.BlockSpec((W, LATP), lambda cc: (cc, 0)),
        pl.BlockSpec((1, LATP), lambda cc: (0, 0)),
        pl.BlockSpec((1, LATP), lambda cc: (0, 0)),
    ],
    out_specs=[
        pl.BlockSpec((N, LATP), lambda cc: (0, 0)),
        pl.BlockSpec((N,), lambda cc: (0,)),
        pl.BlockSpec((N,), lambda cc: (0,)),
    ],
    out_shape=(
        jax.ShapeDtypeStruct((N, LATP), jnp.float32),
        jax.ShapeDtypeStruct((N,), jnp.float32),
        jax.ShapeDtypeStruct((N,), jnp.float32),
    ),
)

_tc3 = pl.pallas_call(
    _tc3_body,
    out_shape=jax.ShapeDtypeStruct((N, LATP), jnp.float32),
)


def kernel(x, edge_index, W1, att_src1, att_dst1, bias1, gamma, beta,
           W2, att_src2, att_dst2, bias2):
    edges = edge_index.reshape(2, NW, NB, K)

    hcat, as1, ad1 = _tc1(x, W1, att_src1, att_dst1)
    out1, den1p = _edge_l1(edges, as1, ad1, hcat)

    w2p = jnp.pad(W2, ((0, 0), (0, LATP - LAT)))
    as2v = jnp.pad(att_src2, ((0, 0), (0, LATP - LAT)))
    ad2v = jnp.pad(att_dst2, ((0, 0), (0, LATP - LAT)))
    bias2p = jnp.pad(bias2, (0, LATP - LAT))

    h2, as2, ad2 = _tc2(out1, den1p.reshape(NW, N), hcat, as1, ad1,
                        gamma.reshape(FP1, 1, W), beta.reshape(FP1, 1, W),
                        bias1.reshape(FP1, 1, W), w2p, as2v, ad2v)
    out2, den2p = _edge_l2(edges, as2, ad2, h2.reshape(1, N, LATP))
    z = _tc3(out2, den2p.reshape(NW, N), h2, as2, ad2, bias2p)
    return z[:, :LAT]
